# baseline (device time: 72795 ns/iter reference)
import jax
import jax.numpy as jnp
from jax import lax
from jax.experimental import pallas as pl
from jax.experimental.pallas import tpu as pltpu

N_DEV = 4
M = 1024
D = 1024
CHUNK = M // N_DEV


def kernel(x, Wg, Wu, Wd):
    def body(x_ref, wg_ref, wu_ref, wd_ref, out_ref,
             rs_buf, send_sems, rs_sems, ag_sems):
        my = lax.axis_index("i")
        right = lax.rem(my + 1, N_DEV)
        left = lax.rem(my + N_DEV - 1, N_DEV)
        diag = lax.rem(my + 2, N_DEV)

        barrier_sem = pltpu.get_barrier_semaphore()
        for nbr in (right, left, diag):
            pl.semaphore_signal(
                barrier_sem, inc=1,
                device_id=(nbr,), device_id_type=pl.DeviceIdType.MESH,
            )
        pl.semaphore_wait(barrier_sem, 3)

        def compute_chunk(c):
            xs = x_ref[pl.ds(c * CHUNK, CHUNK), :]
            g = jnp.dot(xs, wg_ref[...], preferred_element_type=jnp.float32)
            u = jnp.dot(xs, wu_ref[...], preferred_element_type=jnp.float32)
            h = g * (u * lax.logistic(u))
            out_ref[pl.ds(c * CHUNK, CHUNK), :] = jnp.dot(
                h, wd_ref[...], preferred_element_type=jnp.float32)

        def rs_send(target, slot):
            return pltpu.make_async_remote_copy(
                src_ref=out_ref.at[pl.ds(target * CHUNK, CHUNK), :],
                dst_ref=rs_buf.at[slot],
                send_sem=send_sems.at[slot],
                recv_sem=rs_sems.at[slot],
                device_id=(target,),
                device_id_type=pl.DeviceIdType.MESH,
            )

        compute_chunk(diag)
        rs_diag = rs_send(diag, 2)
        rs_diag.start()
        compute_chunk(right)
        rs_right = rs_send(right, 0)
        rs_right.start()
        compute_chunk(left)
        rs_left = rs_send(left, 1)
        rs_left.start()
        compute_chunk(my)

        rs_right.wait_recv()
        rs_left.wait_recv()
        rs_diag.wait_recv()
        own = out_ref[pl.ds(my * CHUNK, CHUNK), :]
        out_ref[pl.ds(my * CHUNK, CHUNK), :] = (
            (own + rs_buf[0]) + (rs_buf[1] + rs_buf[2]))

        def ag_send(target, slot):
            return pltpu.make_async_remote_copy(
                src_ref=out_ref.at[pl.ds(my * CHUNK, CHUNK), :],
                dst_ref=out_ref.at[pl.ds(my * CHUNK, CHUNK), :],
                send_sem=send_sems.at[3 + slot],
                recv_sem=ag_sems.at[slot],
                device_id=(target,),
                device_id_type=pl.DeviceIdType.MESH,
            )

        ag_diag = ag_send(diag, 2)
        ag_diag.start()
        ag_right = ag_send(right, 0)
        ag_right.start()
        ag_left = ag_send(left, 1)
        ag_left.start()

        ag_right.wait_recv()
        ag_left.wait_recv()
        ag_diag.wait_recv()

        rs_right.wait_send()
        rs_left.wait_send()
        rs_diag.wait_send()
        ag_right.wait_send()
        ag_left.wait_send()
        ag_diag.wait_send()

    return pl.pallas_call(
        body,
        out_shape=jax.ShapeDtypeStruct((M, D), jnp.float32),
        in_specs=[
            pl.BlockSpec(memory_space=pltpu.VMEM),
            pl.BlockSpec(memory_space=pltpu.VMEM),
            pl.BlockSpec(memory_space=pltpu.VMEM),
            pl.BlockSpec(memory_space=pltpu.VMEM),
        ],
        out_specs=pl.BlockSpec(memory_space=pltpu.VMEM),
        scratch_shapes=[
            pltpu.VMEM((3, CHUNK, D), jnp.float32),
            pltpu.SemaphoreType.DMA((6,)),
            pltpu.SemaphoreType.DMA((3,)),
            pltpu.SemaphoreType.DMA((3,)),
        ],
        compiler_params=pltpu.CompilerParams(
            collective_id=0,
            vmem_limit_bytes=128 * 1024 * 1024,
        ),
    )(x, Wg, Wu, Wd)


# device time: 72213 ns/iter; 1.0081x vs baseline; 1.0081x over previous
import jax
import jax.numpy as jnp
from jax import lax
from jax.experimental import pallas as pl
from jax.experimental.pallas import tpu as pltpu

N_DEV = 4
M = 1024
D = 1024
CHUNK = M // N_DEV


def kernel(x, Wg, Wu, Wd):
    def body(x_ref, wg_ref, wu_ref, wd_ref, out_ref,
             part_buf, rs_buf, send_sems, rs_sems, ag_sems):
        my = lax.axis_index("i")
        right = lax.rem(my + 1, N_DEV)
        left = lax.rem(my + N_DEV - 1, N_DEV)
        diag = lax.rem(my + 2, N_DEV)

        barrier_sem = pltpu.get_barrier_semaphore()
        for nbr in (right, left, diag):
            pl.semaphore_signal(
                barrier_sem, inc=1,
                device_id=(nbr,), device_id_type=pl.DeviceIdType.MESH,
            )
        pl.semaphore_wait(barrier_sem, 3)

        def compute_into(slot, c):
            xs = x_ref[pl.ds(c * CHUNK, CHUNK), :]
            g = jnp.dot(xs, wg_ref[...], preferred_element_type=jnp.float32)
            u = jnp.dot(xs, wu_ref[...], preferred_element_type=jnp.float32)
            h = g * (u * lax.logistic(u))
            part_buf[slot] = jnp.dot(
                h, wd_ref[...], preferred_element_type=jnp.float32)

        def rs_send(target, src_slot, slot):
            return pltpu.make_async_remote_copy(
                src_ref=part_buf.at[src_slot],
                dst_ref=rs_buf.at[slot],
                send_sem=send_sems.at[slot],
                recv_sem=rs_sems.at[slot],
                device_id=(target,),
                device_id_type=pl.DeviceIdType.MESH,
            )

        compute_into(0, diag)
        rs_diag = rs_send(diag, 0, 2)
        rs_diag.start()
        compute_into(1, right)
        rs_right = rs_send(right, 1, 0)
        rs_right.start()
        compute_into(2, left)
        rs_left = rs_send(left, 2, 1)
        rs_left.start()
        compute_into(3, my)

        rs_right.wait_recv()
        rs_left.wait_recv()
        rs_diag.wait_recv()
        out_ref[pl.ds(my * CHUNK, CHUNK), :] = (
            (part_buf[3] + rs_buf[0]) + (rs_buf[1] + rs_buf[2]))

        def ag_send(target, slot):
            return pltpu.make_async_remote_copy(
                src_ref=out_ref.at[pl.ds(my * CHUNK, CHUNK), :],
                dst_ref=out_ref.at[pl.ds(my * CHUNK, CHUNK), :],
                send_sem=send_sems.at[3 + slot],
                recv_sem=ag_sems.at[slot],
                device_id=(target,),
                device_id_type=pl.DeviceIdType.MESH,
            )

        ag_diag = ag_send(diag, 2)
        ag_diag.start()
        ag_right = ag_send(right, 0)
        ag_right.start()
        ag_left = ag_send(left, 1)
        ag_left.start()

        ag_right.wait_recv()
        ag_left.wait_recv()
        ag_diag.wait_recv()

        rs_right.wait_send()
        rs_left.wait_send()
        rs_diag.wait_send()
        ag_right.wait_send()
        ag_left.wait_send()
        ag_diag.wait_send()

    return pl.pallas_call(
        body,
        out_shape=jax.ShapeDtypeStruct((M, D), jnp.float32),
        in_specs=[
            pl.BlockSpec(memory_space=pltpu.VMEM),
            pl.BlockSpec(memory_space=pltpu.VMEM),
            pl.BlockSpec(memory_space=pltpu.VMEM),
            pl.BlockSpec(memory_space=pltpu.VMEM),
        ],
        out_specs=pl.BlockSpec(memory_space=pltpu.VMEM),
        scratch_shapes=[
            pltpu.VMEM((4, CHUNK, D), jnp.float32),
            pltpu.VMEM((3, CHUNK, D), jnp.float32),
            pltpu.SemaphoreType.DMA((6,)),
            pltpu.SemaphoreType.DMA((3,)),
            pltpu.SemaphoreType.DMA((3,)),
        ],
        compiler_params=pltpu.CompilerParams(
            collective_id=0,
            vmem_limit_bytes=128 * 1024 * 1024,
        ),
    )(x, Wg, Wu, Wd)


# device time: 51676 ns/iter; 1.4087x vs baseline; 1.3974x over previous
import jax
import jax.numpy as jnp
from jax import lax
from jax.experimental import pallas as pl
from jax.experimental.pallas import tpu as pltpu

N_DEV = 4
M = 1024
D = 1024
CHUNK = M // N_DEV


def kernel(x, Wg, Wu, Wd):
    def body(x_ref, wg_ref, wu_ref, wd_ref, out_ref,
             part_buf, own_buf, red_buf, rs_buf, ag_buf,
             send_sems, rs_sems, ag_sems):
        my = lax.axis_index("i")
        right = lax.rem(my + 1, N_DEV)
        left = lax.rem(my + N_DEV - 1, N_DEV)
        diag = lax.rem(my + 2, N_DEV)

        barrier_sem = pltpu.get_barrier_semaphore()
        for nbr in (right, left, diag):
            pl.semaphore_signal(
                barrier_sem, inc=1,
                device_id=(nbr,), device_id_type=pl.DeviceIdType.MESH,
            )
        pl.semaphore_wait(barrier_sem, 3)

        def partial_chunk(c):
            xs = x_ref[pl.ds(c * CHUNK, CHUNK), :]
            g = jnp.dot(xs, wg_ref[...], preferred_element_type=jnp.float32)
            u = jnp.dot(xs, wu_ref[...], preferred_element_type=jnp.float32)
            h = g * (u * lax.logistic(u))
            return jnp.dot(h, wd_ref[...], preferred_element_type=jnp.float32)

        def rs_send(target, src_slot, slot):
            return pltpu.make_async_remote_copy(
                src_ref=part_buf.at[src_slot],
                dst_ref=rs_buf.at[slot],
                send_sem=send_sems.at[slot],
                recv_sem=rs_sems.at[slot],
                device_id=(target,),
                device_id_type=pl.DeviceIdType.MESH,
            )

        part_buf[0] = partial_chunk(diag).astype(jnp.bfloat16)
        rs_diag = rs_send(diag, 0, 2)
        rs_diag.start()
        part_buf[1] = partial_chunk(right).astype(jnp.bfloat16)
        rs_right = rs_send(right, 1, 0)
        rs_right.start()
        part_buf[2] = partial_chunk(left).astype(jnp.bfloat16)
        rs_left = rs_send(left, 2, 1)
        rs_left.start()
        own_buf[...] = partial_chunk(my)

        rs_right.wait_recv()
        rs_left.wait_recv()
        rs_diag.wait_recv()
        red = (own_buf[...] + rs_buf[0].astype(jnp.float32)) + (
            rs_buf[1].astype(jnp.float32) + rs_buf[2].astype(jnp.float32))
        out_ref[pl.ds(my * CHUNK, CHUNK), :] = red
        red_buf[...] = red.astype(jnp.bfloat16)

        def ag_send(target, slot):
            return pltpu.make_async_remote_copy(
                src_ref=red_buf,
                dst_ref=ag_buf.at[slot],
                send_sem=send_sems.at[3 + slot],
                recv_sem=ag_sems.at[slot],
                device_id=(target,),
                device_id_type=pl.DeviceIdType.MESH,
            )

        ag_diag = ag_send(diag, 2)
        ag_diag.start()
        ag_right = ag_send(right, 0)
        ag_right.start()
        ag_left = ag_send(left, 1)
        ag_left.start()

        ag_right.wait_recv()
        out_ref[pl.ds(left * CHUNK, CHUNK), :] = ag_buf[0].astype(jnp.float32)
        ag_left.wait_recv()
        out_ref[pl.ds(right * CHUNK, CHUNK), :] = ag_buf[1].astype(jnp.float32)
        ag_diag.wait_recv()
        out_ref[pl.ds(diag * CHUNK, CHUNK), :] = ag_buf[2].astype(jnp.float32)

        rs_right.wait_send()
        rs_left.wait_send()
        rs_diag.wait_send()
        ag_right.wait_send()
        ag_left.wait_send()
        ag_diag.wait_send()

    return pl.pallas_call(
        body,
        out_shape=jax.ShapeDtypeStruct((M, D), jnp.float32),
        in_specs=[
            pl.BlockSpec(memory_space=pltpu.VMEM),
            pl.BlockSpec(memory_space=pltpu.VMEM),
            pl.BlockSpec(memory_space=pltpu.VMEM),
            pl.BlockSpec(memory_space=pltpu.VMEM),
        ],
        out_specs=pl.BlockSpec(memory_space=pltpu.VMEM),
        scratch_shapes=[
            pltpu.VMEM((3, CHUNK, D), jnp.bfloat16),
            pltpu.VMEM((CHUNK, D), jnp.float32),
            pltpu.VMEM((CHUNK, D), jnp.bfloat16),
            pltpu.VMEM((3, CHUNK, D), jnp.bfloat16),
            pltpu.VMEM((3, CHUNK, D), jnp.bfloat16),
            pltpu.SemaphoreType.DMA((6,)),
            pltpu.SemaphoreType.DMA((3,)),
            pltpu.SemaphoreType.DMA((3,)),
        ],
        compiler_params=pltpu.CompilerParams(
            collective_id=0,
            vmem_limit_bytes=128 * 1024 * 1024,
        ),
    )(x, Wg, Wu, Wd)
